# Initial kernel scaffold; baseline (speedup 1.0000x reference)
import jax
import jax.numpy as jnp
from jax.experimental import pallas as pl


def kernel(x, edge_index, W1, b1, W2, b2, W_ih, W_hh, b_ih, b_hh, Wl, bl):
    raise NotImplementedError("write your pallas kernel here")


# speedup vs baseline: 7.6570x; 7.6570x over previous
import functools
import jax
import jax.numpy as jnp
from jax import lax
from jax.experimental import pallas as pl
from jax.experimental.pallas import tpu as pltpu
from jax.experimental.pallas import tpu_sc as plsc

N_NODES = 10000
NPAD = 10240
D_IN = 256
H = 512
G4 = 4 * H
N_CLASSES = 40
N_EDGES = 160000
EPAD = 163840

NC, NS = 2, 16
ROWS_PER_SUB = NPAD // NS
def _dot(a, b):
    return jnp.dot(a, b, preferred_element_type=jnp.float32)


def _sc_mesh():
    return plsc.VectorSubcoreMesh(core_axis_name="c", subcore_axis_name="s",
                                  num_cores=NC, num_subcores=NS)


def _deg_partials(dstp):
    e_per_core = EPAD // NC
    e_per_sub = e_per_core // NS
    n_chunks = e_per_sub // 128

    @functools.partial(
        pl.kernel,
        out_type=jax.ShapeDtypeStruct((NC, NPAD), jnp.float32),
        mesh=_sc_mesh(),
        scratch_types=[
            pltpu.VMEM((128,), jnp.int32),
            pltpu.VMEM((128,), jnp.float32),
            pltpu.VMEM((ROWS_PER_SUB,), jnp.float32),
            pltpu.VMEM_SHARED((NPAD,), jnp.float32),
        ],
    )
    def k(dst_hbm, out_hbm, idx_v, ones_v, zero_v, acc_sh):
        cid = lax.axis_index("c")
        sid = lax.axis_index("s")
        for i in range(128 // 16):
            ones_v[pl.ds(i * 16, 16)] = jnp.ones((16,), jnp.float32)
        def zb(i, carry):
            zero_v[pl.ds(i * 16, 16)] = jnp.zeros((16,), jnp.float32)
            return carry
        lax.fori_loop(0, ROWS_PER_SUB // 16, zb, 0)
        pltpu.sync_copy(zero_v, acc_sh.at[pl.ds(sid * ROWS_PER_SUB, ROWS_PER_SUB)])
        plsc.subcore_barrier()
        base = cid * e_per_core + sid * e_per_sub
        def ch(j, carry):
            pltpu.sync_copy(dst_hbm.at[pl.ds(base + j * 128, 128)], idx_v)
            pltpu.sync_copy(ones_v, acc_sh.at[idx_v], add=True)
            return carry
        lax.fori_loop(0, n_chunks, ch, 0)
        plsc.subcore_barrier()
        pltpu.sync_copy(acc_sh.at[pl.ds(sid * ROWS_PER_SUB, ROWS_PER_SUB)],
                        out_hbm.at[cid, pl.ds(sid * ROWS_PER_SUB, ROWS_PER_SUB)])

    return k(dstp)


def _neighbor_sum(xq0, xq1, xq2, xq3, src2d, dst2d):
    chunks_per_sub = (EPAD // NS) // 128
    blocks_per_sub = chunks_per_sub // 8

    @functools.partial(
        pl.kernel,
        out_type=jax.ShapeDtypeStruct((NPAD, 4 * 128), jnp.float32),
        mesh=_sc_mesh(),
        scratch_types=[
            pltpu.VMEM((8, 128), jnp.int32),
            pltpu.VMEM((8, 128), jnp.int32),
            pltpu.VMEM((128, 128), jnp.float32),
            pltpu.VMEM((128, 128), jnp.float32),
            pltpu.VMEM_SHARED((NPAD, 128), jnp.float32),
            pltpu.SemaphoreType.DMA,
            pltpu.SemaphoreType.DMA,
            pltpu.SemaphoreType.DMA,
            pltpu.SemaphoreType.DMA,
        ],
    )
    def k(x0, x1, x2, x3, src_hbm, dst_hbm, out_hbm,
          src_blk, dst_blk, rows0, rows1, acc_sh,
          sem_g0, sem_g1, sem_s0, sem_s1):
        cid = lax.axis_index("c")
        sid = lax.axis_index("s")
        r0 = sid * ROWS_PER_SUB
        rows = (rows0, rows1)
        sem_g = (sem_g0, sem_g1)
        sem_s = (sem_s0, sem_s1)

        def quarter(x_ref, q):
            pltpu.sync_copy(x_ref.at[pl.ds(r0, ROWS_PER_SUB)],
                            acc_sh.at[pl.ds(r0, ROWS_PER_SUB)])
            plsc.subcore_barrier()
            c0 = sid * chunks_per_sub

            def blk(jj, carry):
                @pl.when(jj > 0)
                def _():
                    pltpu.make_async_copy(
                        rows0, acc_sh.at[dst_blk.at[6]], sem_s0).wait()
                    pltpu.make_async_copy(
                        rows1, acc_sh.at[dst_blk.at[7]], sem_s1).wait()
                pltpu.sync_copy(src_hbm.at[pl.ds(c0 + jj * 8, 8)], src_blk)
                pltpu.sync_copy(dst_hbm.at[pl.ds(c0 + jj * 8, 8)], dst_blk)
                descs = {}
                for b in range(8):
                    B = b % 2
                    if b >= 2:
                        descs[b - 2].wait()
                    pltpu.async_copy(
                        x_ref.at[src_blk.at[b]], rows[B], sem_g[B]).wait()
                    descs[b] = pltpu.async_copy(
                        rows[B], acc_sh.at[dst_blk.at[b]], sem_s[B], add=True)
                return carry
            lax.fori_loop(0, blocks_per_sub, blk, 0)
            pltpu.make_async_copy(rows0, acc_sh.at[dst_blk.at[6]], sem_s0).wait()
            pltpu.make_async_copy(rows1, acc_sh.at[dst_blk.at[7]], sem_s1).wait()
            plsc.subcore_barrier()
            pltpu.sync_copy(acc_sh.at[pl.ds(r0, ROWS_PER_SUB)],
                            out_hbm.at[pl.ds(r0, ROWS_PER_SUB),
                                       pl.ds(q * 128, 128)])

        @pl.when(cid == 0)
        def _():
            quarter(x0, 0)
            quarter(x1, 1)

        @pl.when(cid == 1)
        def _():
            quarter(x2, 2)
            quarter(x3, 3)

    return k(xq0, xq1, xq2, xq3, src2d, dst2d)


def _dinv_kernel(partials):
    def body(p_ref, o_ref):
        deg = p_ref[0:1, :] + p_ref[1:2, :] + 1.0
        o_ref[...] = lax.rsqrt(deg)
    return pl.pallas_call(
        body,
        out_shape=jax.ShapeDtypeStruct((1, NPAD), jnp.float32),
    )(partials)


def _xw_quarters(xp, W, dinv_col, bm):
    m_blocks = NPAD // bm
    d_in = W.shape[0]

    def body2(x_ref, w_ref, dinv_ref, o_ref):
        o_ref[0, ...] = dinv_ref[...] * _dot(x_ref[...], w_ref[...])

    out = pl.pallas_call(
        body2,
        grid=(4, m_blocks),
        in_specs=[
            pl.BlockSpec((bm, d_in), lambda q, i: (i, 0)),
            pl.BlockSpec((d_in, 128), lambda q, i: (0, q)),
            pl.BlockSpec((bm, 1), lambda q, i: (i, 0)),
        ],
        out_specs=pl.BlockSpec((1, bm, 128), lambda q, i: (q, i, 0)),
        out_shape=jax.ShapeDtypeStruct((4, NPAD, 128), jnp.float32),
    )(xp, W, dinv_col)
    return out[0], out[1], out[2], out[3]


def _conv_finish_matmul(comb, dinv_col, b_prev, W, bm, scale_out):
    m_blocks = NPAD // bm

    def body(comb_ref, dinv_ref, b_ref, w_ref, o_ref, h_s):
        q = pl.program_id(1)
        @pl.when(q == 0)
        def _():
            h_s[...] = jnp.maximum(dinv_ref[...] * comb_ref[...] + b_ref[...],
                                   0.0)
        res = _dot(h_s[...], w_ref[...])
        if scale_out:
            res = dinv_ref[...] * res
        o_ref[0, ...] = res

    out = pl.pallas_call(
        body,
        grid=(m_blocks, 4),
        in_specs=[
            pl.BlockSpec((bm, H), lambda i, q: (i, 0)),
            pl.BlockSpec((bm, 1), lambda i, q: (i, 0)),
            pl.BlockSpec((1, H), lambda i, q: (0, 0)),
            pl.BlockSpec((H, 128), lambda i, q: (0, q)),
        ],
        out_specs=pl.BlockSpec((1, bm, 128), lambda i, q: (q, i, 0)),
        out_shape=jax.ShapeDtypeStruct((4, NPAD, 128), jnp.float32),
        scratch_shapes=[pltpu.VMEM((bm, H), jnp.float32)],
    )(comb, dinv_col, b_prev, W)
    return out[0], out[1], out[2], out[3]


def _gates_matmul(comb2, dinv_col, b2, W_ihT, bias, bm):
    m_blocks = NPAD // bm

    def body(comb_ref, dinv_ref, b2_ref, w_ref, bias_ref, o_ref):
        h2 = jnp.maximum(dinv_ref[...] * comb_ref[...] + b2_ref[...], 0.0)
        o_ref[...] = _dot(h2, w_ref[...]) + bias_ref[...]

    return pl.pallas_call(
        body,
        grid=(m_blocks,),
        in_specs=[
            pl.BlockSpec((bm, H), lambda i: (i, 0)),
            pl.BlockSpec((bm, 1), lambda i: (i, 0)),
            pl.BlockSpec((1, H), lambda i: (0, 0)),
            pl.BlockSpec((H, G4), lambda i: (0, 0)),
            pl.BlockSpec((1, G4), lambda i: (0, 0)),
        ],
        out_specs=pl.BlockSpec((bm, G4), lambda i: (i, 0)),
        out_shape=jax.ShapeDtypeStruct((NPAD, G4), jnp.float32),
    )(comb2, dinv_col, b2, W_ihT, bias)


def _lstm(gates_pre, W_hhT, bt):
    t_blocks = NPAD // bt

    def body(g_ref, w_ref, o_ref, h_s, c_s):
        @pl.when(pl.program_id(0) == 0)
        def _():
            h_s[...] = jnp.zeros_like(h_s)
            c_s[...] = jnp.zeros_like(c_s)

        def step(t, carry):
            g = g_ref[pl.ds(t, 1), :] + jnp.dot(
                h_s[...].astype(jnp.bfloat16), w_ref[...],
                preferred_element_type=jnp.float32)
            i_g = jax.nn.sigmoid(g[:, 0 * H:1 * H])
            f_g = jax.nn.sigmoid(g[:, 1 * H:2 * H])
            g_g = jnp.tanh(g[:, 2 * H:3 * H])
            o_g = jax.nn.sigmoid(g[:, 3 * H:4 * H])
            c = f_g * c_s[...] + i_g * g_g
            h = o_g * jnp.tanh(c)
            c_s[...] = c
            h_s[...] = h
            o_ref[pl.ds(t, 1), :] = h
            return carry
        lax.fori_loop(0, bt, step, 0, unroll=4)

    return pl.pallas_call(
        body,
        grid=(t_blocks,),
        in_specs=[
            pl.BlockSpec((bt, G4), lambda i: (i, 0)),
            pl.BlockSpec((H, G4), lambda i: (0, 0)),
        ],
        out_specs=pl.BlockSpec((bt, H), lambda i: (i, 0)),
        out_shape=jax.ShapeDtypeStruct((NPAD, H), jnp.float32),
        scratch_shapes=[pltpu.VMEM((1, H), jnp.float32),
                        pltpu.VMEM((1, H), jnp.float32)],
    )(gates_pre, W_hhT)


def _head(hs, Wl_pad, bl_pad, bm):
    m_blocks = NPAD // bm

    def body(h_ref, w_ref, b_ref, o_ref):
        logits = _dot(h_ref[...], w_ref[...]) + b_ref[...]
        m = jnp.max(logits, axis=1, keepdims=True)
        lse = jnp.log(jnp.sum(jnp.exp(logits - m), axis=1, keepdims=True))
        o_ref[...] = logits - m - lse

    return pl.pallas_call(
        body,
        grid=(m_blocks,),
        in_specs=[
            pl.BlockSpec((bm, H), lambda i: (i, 0)),
            pl.BlockSpec((H, 128), lambda i: (0, 0)),
            pl.BlockSpec((1, 128), lambda i: (0, 0)),
        ],
        out_specs=pl.BlockSpec((bm, 128), lambda i: (i, 0)),
        out_shape=jax.ShapeDtypeStruct((NPAD, 128), jnp.float32),
    )(hs, Wl_pad, bl_pad)


def kernel(x, edge_index, W1, b1, W2, b2, W_ih, W_hh, b_ih, b_hh, Wl, bl):
    f32 = jnp.float32
    xp = jnp.pad(x.astype(f32), ((0, NPAD - N_NODES), (0, 0)))
    pad_e = jnp.full((EPAD - N_EDGES,), NPAD - 1, jnp.int32)
    srcp = jnp.concatenate([edge_index[0].astype(jnp.int32), pad_e])
    dstp = jnp.concatenate([edge_index[1].astype(jnp.int32), pad_e])
    b1r = b1.reshape(1, H).astype(f32)
    b2r = b2.reshape(1, H).astype(f32)
    W_ihT = W_ih.T.astype(f32)
    W_hhT = W_hh.T.astype(jnp.bfloat16)
    bias = (b_ih + b_hh).reshape(1, G4).astype(f32)
    Wl_pad = jnp.pad(Wl.astype(f32), ((0, 0), (0, 128 - N_CLASSES)))
    bl_pad = jnp.concatenate(
        [bl.astype(f32), jnp.full((128 - N_CLASSES,), -1e30, f32)]
    ).reshape(1, 128)

    partials = _deg_partials(dstp)
    dinv = _dinv_kernel(partials)
    dinv_col = dinv.reshape(NPAD, 1)

    src2d = srcp.reshape(EPAD // 128, 128)
    dst2d = dstp.reshape(EPAD // 128, 128)
    q0, q1, q2, q3 = _xw_quarters(xp, W1.astype(f32), dinv_col, bm=1024)
    comb1 = _neighbor_sum(q0, q1, q2, q3, src2d, dst2d)
    r0, r1, r2, r3 = _conv_finish_matmul(comb1, dinv_col, b1r,
                                         W2.astype(f32), bm=1024,
                                         scale_out=True)
    comb2 = _neighbor_sum(r0, r1, r2, r3, src2d, dst2d)
    gates_pre = _gates_matmul(comb2, dinv_col, b2r, W_ihT, bias, bm=512)
    hs = _lstm(gates_pre, W_hhT, bt=128)
    out = _head(hs, Wl_pad, bl_pad, bm=1024)
    return out[:N_NODES, :N_CLASSES]


# speedup vs baseline: 7.9357x; 1.0364x over previous
import functools
import jax
import jax.numpy as jnp
from jax import lax
from jax.experimental import pallas as pl
from jax.experimental.pallas import tpu as pltpu
from jax.experimental.pallas import tpu_sc as plsc

N_NODES = 10000
NPAD = 10240
D_IN = 256
H = 512
G4 = 4 * H
N_CLASSES = 40
N_EDGES = 160000
EPAD = 163840

NC, NS = 2, 16
ROWS_PER_SUB = NPAD // NS
def _dot(a, b):
    return jnp.dot(a, b, preferred_element_type=jnp.float32)


def _sc_mesh():
    return plsc.VectorSubcoreMesh(core_axis_name="c", subcore_axis_name="s",
                                  num_cores=NC, num_subcores=NS)


def _deg_partials(dstp):
    e_per_core = EPAD // NC
    e_per_sub = e_per_core // NS
    n_chunks = e_per_sub // 128

    @functools.partial(
        pl.kernel,
        out_type=jax.ShapeDtypeStruct((NC, NPAD), jnp.float32),
        mesh=_sc_mesh(),
        scratch_types=[
            pltpu.VMEM((128,), jnp.int32),
            pltpu.VMEM((128,), jnp.float32),
            pltpu.VMEM((ROWS_PER_SUB,), jnp.float32),
            pltpu.VMEM_SHARED((NPAD,), jnp.float32),
        ],
    )
    def k(dst_hbm, out_hbm, idx_v, ones_v, zero_v, acc_sh):
        cid = lax.axis_index("c")
        sid = lax.axis_index("s")
        for i in range(128 // 16):
            ones_v[pl.ds(i * 16, 16)] = jnp.ones((16,), jnp.float32)
        def zb(i, carry):
            zero_v[pl.ds(i * 16, 16)] = jnp.zeros((16,), jnp.float32)
            return carry
        lax.fori_loop(0, ROWS_PER_SUB // 16, zb, 0)
        pltpu.sync_copy(zero_v, acc_sh.at[pl.ds(sid * ROWS_PER_SUB, ROWS_PER_SUB)])
        plsc.subcore_barrier()
        base = cid * e_per_core + sid * e_per_sub
        def ch(j, carry):
            pltpu.sync_copy(dst_hbm.at[pl.ds(base + j * 128, 128)], idx_v)
            pltpu.sync_copy(ones_v, acc_sh.at[idx_v], add=True)
            return carry
        lax.fori_loop(0, n_chunks, ch, 0)
        plsc.subcore_barrier()
        pltpu.sync_copy(acc_sh.at[pl.ds(sid * ROWS_PER_SUB, ROWS_PER_SUB)],
                        out_hbm.at[cid, pl.ds(sid * ROWS_PER_SUB, ROWS_PER_SUB)])

    return k(dstp)


def _neighbor_sum(xq0, xq1, xq2, xq3, src2d, dst2d):
    CH = 80
    chunks_per_sub = (EPAD // NS) // CH
    n_super = chunks_per_sub // 16

    @functools.partial(
        pl.kernel,
        out_type=jax.ShapeDtypeStruct((NPAD, 4 * 128), jnp.float32),
        mesh=_sc_mesh(),
        scratch_types=[
            pltpu.VMEM((2, 8, 80), jnp.int32),
            pltpu.VMEM((2, 8, 80), jnp.int32),
            pltpu.VMEM((80, 128), jnp.float32),
            pltpu.VMEM((80, 128), jnp.float32),
            pltpu.VMEM((80, 128), jnp.float32),
            pltpu.VMEM((80, 128), jnp.float32),
            pltpu.VMEM_SHARED((NPAD, 128), jnp.float32),
            pltpu.SemaphoreType.DMA,
            pltpu.SemaphoreType.DMA,
            pltpu.SemaphoreType.DMA,
            pltpu.SemaphoreType.DMA,
            pltpu.SemaphoreType.DMA,
            pltpu.SemaphoreType.DMA,
            pltpu.SemaphoreType.DMA,
            pltpu.SemaphoreType.DMA,
        ],
    )
    def k(x0, x1, x2, x3, src_hbm, dst_hbm, out_hbm,
          src_ix, dst_ix, rows0, rows1, rows2, rows3, acc_sh,
          sg0, sg1, sg2, sg3, ss0, ss1, ss2, ss3):
        cid = lax.axis_index("c")
        sid = lax.axis_index("s")
        r0 = sid * ROWS_PER_SUB
        rows = (rows0, rows1, rows2, rows3)
        sem_g = (sg0, sg1, sg2, sg3)
        sem_s = (ss0, ss1, ss2, ss3)

        def quarter(x_ref, q):
            pltpu.sync_copy(x_ref.at[pl.ds(r0, ROWS_PER_SUB)],
                            acc_sh.at[pl.ds(r0, ROWS_PER_SUB)])
            plsc.subcore_barrier()
            c0 = sid * chunks_per_sub

            def fire_g(p, b, B):
                pltpu.async_copy(x_ref.at[src_ix.at[p, b]], rows[B], sem_g[B])

            def fire_s(p, b, B):
                return pltpu.async_copy(rows[B], acc_sh.at[dst_ix.at[p, b]],
                                        sem_s[B], add=True)

            def wait_s(p, b, B):
                pltpu.make_async_copy(rows[B], acc_sh.at[dst_ix.at[p, b]],
                                      sem_s[B]).wait()

            def wait_g(p, b, B):
                pltpu.make_async_copy(x_ref.at[src_ix.at[p, b]], rows[B],
                                      sem_g[B]).wait()

            pltpu.sync_copy(src_hbm.at[pl.ds(c0, 8)], src_ix.at[0])
            pltpu.sync_copy(dst_hbm.at[pl.ds(c0, 8)], dst_ix.at[0])
            pltpu.sync_copy(src_hbm.at[pl.ds(c0 + 8, 8)], src_ix.at[1])
            pltpu.sync_copy(dst_hbm.at[pl.ds(c0 + 8, 8)], dst_ix.at[1])
            fire_g(0, 0, 0)
            fire_g(0, 1, 1)

            def sup(J, carry):
                for p in range(2):
                    for b in range(8):
                        c = p * 8 + b
                        B = c % 4
                        B2 = (c + 2) % 4
                        p2, b2 = divmod((c - 2) % 16, 8)
                        @pl.when((J > 0) | (c >= 2))
                        def _():
                            wait_s(p2, b2, B2)
                        pn, bn = divmod((c + 2) % 16, 8)
                        gc = J * 16 + c + 2
                        @pl.when(gc < chunks_per_sub)
                        def _():
                            fire_g(pn, bn, B2)
                        wait_g(p, b, B)
                        fire_s(p, b, B)
                        if b == 3:
                            nxt = c0 + (J + p) * 16 + (1 - p) * 8
                            @pl.when(J + p < n_super)
                            def _():
                                pltpu.sync_copy(src_hbm.at[pl.ds(nxt, 8)],
                                                src_ix.at[1 - p])
                                pltpu.sync_copy(dst_hbm.at[pl.ds(nxt, 8)],
                                                dst_ix.at[1 - p])
                return carry
            lax.fori_loop(0, n_super, sup, 0)
            wait_s(1, 6, 2)
            wait_s(1, 7, 3)
            plsc.subcore_barrier()
            pltpu.sync_copy(acc_sh.at[pl.ds(r0, ROWS_PER_SUB)],
                            out_hbm.at[pl.ds(r0, ROWS_PER_SUB),
                                       pl.ds(q * 128, 128)])

        @pl.when(cid == 0)
        def _():
            quarter(x0, 0)
            quarter(x1, 1)

        @pl.when(cid == 1)
        def _():
            quarter(x2, 2)
            quarter(x3, 3)

    return k(xq0, xq1, xq2, xq3, src2d, dst2d)


def _dinv_kernel(partials):
    def body(p_ref, o_ref):
        deg = p_ref[0:1, :] + p_ref[1:2, :] + 1.0
        o_ref[...] = lax.rsqrt(deg)
    return pl.pallas_call(
        body,
        out_shape=jax.ShapeDtypeStruct((1, NPAD), jnp.float32),
    )(partials)


def _xw_quarters(xp, W, dinv_col, bm):
    m_blocks = NPAD // bm
    d_in = W.shape[0]

    def body2(x_ref, w_ref, dinv_ref, o_ref):
        o_ref[0, ...] = dinv_ref[...] * _dot(x_ref[...], w_ref[...])

    out = pl.pallas_call(
        body2,
        grid=(4, m_blocks),
        in_specs=[
            pl.BlockSpec((bm, d_in), lambda q, i: (i, 0)),
            pl.BlockSpec((d_in, 128), lambda q, i: (0, q)),
            pl.BlockSpec((bm, 1), lambda q, i: (i, 0)),
        ],
        out_specs=pl.BlockSpec((1, bm, 128), lambda q, i: (q, i, 0)),
        out_shape=jax.ShapeDtypeStruct((4, NPAD, 128), jnp.float32),
    )(xp, W, dinv_col)
    return out[0], out[1], out[2], out[3]


def _conv_finish_matmul(comb, dinv_col, b_prev, W, bm, scale_out):
    m_blocks = NPAD // bm

    def body(comb_ref, dinv_ref, b_ref, w_ref, o_ref, h_s):
        q = pl.program_id(1)
        @pl.when(q == 0)
        def _():
            h_s[...] = jnp.maximum(dinv_ref[...] * comb_ref[...] + b_ref[...],
                                   0.0)
        res = _dot(h_s[...], w_ref[...])
        if scale_out:
            res = dinv_ref[...] * res
        o_ref[0, ...] = res

    out = pl.pallas_call(
        body,
        grid=(m_blocks, 4),
        in_specs=[
            pl.BlockSpec((bm, H), lambda i, q: (i, 0)),
            pl.BlockSpec((bm, 1), lambda i, q: (i, 0)),
            pl.BlockSpec((1, H), lambda i, q: (0, 0)),
            pl.BlockSpec((H, 128), lambda i, q: (0, q)),
        ],
        out_specs=pl.BlockSpec((1, bm, 128), lambda i, q: (q, i, 0)),
        out_shape=jax.ShapeDtypeStruct((4, NPAD, 128), jnp.float32),
        scratch_shapes=[pltpu.VMEM((bm, H), jnp.float32)],
    )(comb, dinv_col, b_prev, W)
    return out[0], out[1], out[2], out[3]


def _gates_matmul(comb2, dinv_col, b2, W_ihT, bias, bm):
    m_blocks = NPAD // bm

    def body(comb_ref, dinv_ref, b2_ref, w_ref, bias_ref, o_ref):
        h2 = jnp.maximum(dinv_ref[...] * comb_ref[...] + b2_ref[...], 0.0)
        o_ref[...] = _dot(h2, w_ref[...]) + bias_ref[...]

    return pl.pallas_call(
        body,
        grid=(m_blocks,),
        in_specs=[
            pl.BlockSpec((bm, H), lambda i: (i, 0)),
            pl.BlockSpec((bm, 1), lambda i: (i, 0)),
            pl.BlockSpec((1, H), lambda i: (0, 0)),
            pl.BlockSpec((H, G4), lambda i: (0, 0)),
            pl.BlockSpec((1, G4), lambda i: (0, 0)),
        ],
        out_specs=pl.BlockSpec((bm, G4), lambda i: (i, 0)),
        out_shape=jax.ShapeDtypeStruct((NPAD, G4), jnp.float32),
    )(comb2, dinv_col, b2, W_ihT, bias)


def _lstm(gates_pre, W_hhT, bt):
    t_blocks = NPAD // bt

    def body(g_ref, w_ref, o_ref, h_s, c_s):
        @pl.when(pl.program_id(0) == 0)
        def _():
            h_s[...] = jnp.zeros_like(h_s)
            c_s[...] = jnp.zeros_like(c_s)

        def step(t, carry):
            g = g_ref[pl.ds(t, 1), :] + jnp.dot(
                h_s[...].astype(jnp.bfloat16), w_ref[...],
                preferred_element_type=jnp.float32)
            i_g = jax.nn.sigmoid(g[:, 0 * H:1 * H])
            f_g = jax.nn.sigmoid(g[:, 1 * H:2 * H])
            g_g = jnp.tanh(g[:, 2 * H:3 * H])
            o_g = jax.nn.sigmoid(g[:, 3 * H:4 * H])
            c = f_g * c_s[...] + i_g * g_g
            h = o_g * jnp.tanh(c)
            c_s[...] = c
            h_s[...] = h
            o_ref[pl.ds(t, 1), :] = h
            return carry
        lax.fori_loop(0, bt, step, 0, unroll=4)

    return pl.pallas_call(
        body,
        grid=(t_blocks,),
        in_specs=[
            pl.BlockSpec((bt, G4), lambda i: (i, 0)),
            pl.BlockSpec((H, G4), lambda i: (0, 0)),
        ],
        out_specs=pl.BlockSpec((bt, H), lambda i: (i, 0)),
        out_shape=jax.ShapeDtypeStruct((NPAD, H), jnp.float32),
        scratch_shapes=[pltpu.VMEM((1, H), jnp.float32),
                        pltpu.VMEM((1, H), jnp.float32)],
    )(gates_pre, W_hhT)


def _head(hs, Wl_pad, bl_pad, bm):
    m_blocks = NPAD // bm

    def body(h_ref, w_ref, b_ref, o_ref):
        logits = _dot(h_ref[...], w_ref[...]) + b_ref[...]
        m = jnp.max(logits, axis=1, keepdims=True)
        lse = jnp.log(jnp.sum(jnp.exp(logits - m), axis=1, keepdims=True))
        o_ref[...] = logits - m - lse

    return pl.pallas_call(
        body,
        grid=(m_blocks,),
        in_specs=[
            pl.BlockSpec((bm, H), lambda i: (i, 0)),
            pl.BlockSpec((H, 128), lambda i: (0, 0)),
            pl.BlockSpec((1, 128), lambda i: (0, 0)),
        ],
        out_specs=pl.BlockSpec((bm, 128), lambda i: (i, 0)),
        out_shape=jax.ShapeDtypeStruct((NPAD, 128), jnp.float32),
    )(hs, Wl_pad, bl_pad)


def kernel(x, edge_index, W1, b1, W2, b2, W_ih, W_hh, b_ih, b_hh, Wl, bl):
    f32 = jnp.float32
    xp = jnp.pad(x.astype(f32), ((0, NPAD - N_NODES), (0, 0)))
    pad_e = jnp.full((EPAD - N_EDGES,), NPAD - 1, jnp.int32)
    srcp = jnp.concatenate([edge_index[0].astype(jnp.int32), pad_e])
    dstp = jnp.concatenate([edge_index[1].astype(jnp.int32), pad_e])
    b1r = b1.reshape(1, H).astype(f32)
    b2r = b2.reshape(1, H).astype(f32)
    W_ihT = W_ih.T.astype(f32)
    W_hhT = W_hh.T.astype(jnp.bfloat16)
    bias = (b_ih + b_hh).reshape(1, G4).astype(f32)
    Wl_pad = jnp.pad(Wl.astype(f32), ((0, 0), (0, 128 - N_CLASSES)))
    bl_pad = jnp.concatenate(
        [bl.astype(f32), jnp.full((128 - N_CLASSES,), -1e30, f32)]
    ).reshape(1, 128)

    partials = _deg_partials(dstp)
    dinv = _dinv_kernel(partials)
    dinv_col = dinv.reshape(NPAD, 1)

    src2d = srcp.reshape(EPAD // 80, 80)
    dst2d = dstp.reshape(EPAD // 80, 80)
    q0, q1, q2, q3 = _xw_quarters(xp, W1.astype(f32), dinv_col, bm=1024)
    comb1 = _neighbor_sum(q0, q1, q2, q3, src2d, dst2d)
    r0, r1, r2, r3 = _conv_finish_matmul(comb1, dinv_col, b1r,
                                         W2.astype(f32), bm=1024,
                                         scale_out=True)
    comb2 = _neighbor_sum(r0, r1, r2, r3, src2d, dst2d)
    gates_pre = _gates_matmul(comb2, dinv_col, b2r, W_ihT, bias, bm=512)
    hs = _lstm(gates_pre, W_hhT, bt=128)
    out = _head(hs, Wl_pad, bl_pad, bm=1024)
    return out[:N_NODES, :N_CLASSES]
